# Initial kernel scaffold; baseline (speedup 1.0000x reference)
import jax
import jax.numpy as jnp
from jax.experimental import pallas as pl


def kernel(boxes, scores):
    raise NotImplementedError("write your pallas kernel here")


# speedup vs baseline: 11.4685x; 11.4685x over previous
import jax
import jax.numpy as jnp
from jax import lax
from jax.experimental import pallas as pl
from jax.experimental.pallas import tpu as pltpu

IOU_T = 0.5
SCORE_T = 0.05
MAXDET = 100
NEG = -1e9

B, N, C = 8, 20000, 91
NP = 20480
CH = 2000
NCH = N // CH


def _class_reduce_body(s_ref, m_ref, c_ref):
    x = s_ref[0]
    m = jnp.max(x, axis=1)
    it = lax.broadcasted_iota(jnp.int32, (CH, C), 1)
    c = jnp.min(jnp.where(x == m[:, None], it, C), axis=1)
    m_ref[0, :, 0] = m
    c_ref[0, :, 0] = c


def _nms_body(sm_ref, cl_ref, y1_ref, x1_ref, y2_ref, x2_ref,
              osc_ref, oy1_ref, ox1_ref, oy2_ref, ox2_ref, ocl_ref, ovd_ref):
    smax = sm_ref[...]
    cls = cl_ref[...]
    y1 = y1_ref[...]
    x1 = x1_ref[...]
    y2 = y2_ref[...]
    x2 = x2_ref[...]
    area = (y2 - y1) * (x2 - x1)
    lane = lax.broadcasted_iota(jnp.int32, (B, NP), 1)
    slot_lane = lax.broadcasted_iota(jnp.int32, (B, 128), 1)

    live0 = jnp.where(smax >= SCORE_T, smax, NEG)
    zf = jnp.full((B, 128), -1.0, jnp.float32)
    zi = jnp.full((B, 128), -1, jnp.int32)
    nv0 = jnp.zeros((B, 1), jnp.int32)

    def body(t, carry):
        live, osc, oy1, ox1, oy2, ox2, ocl, nv = carry
        m = jnp.max(live, axis=1, keepdims=True)
        valid = m > (NEG / 2)
        ismax = live == m
        pick = jnp.min(jnp.where(ismax, lane, NP), axis=1, keepdims=True)
        sel = lane == pick
        by1 = jnp.sum(jnp.where(sel, y1, 0.0), axis=1, keepdims=True)
        bx1 = jnp.sum(jnp.where(sel, x1, 0.0), axis=1, keepdims=True)
        by2 = jnp.sum(jnp.where(sel, y2, 0.0), axis=1, keepdims=True)
        bx2 = jnp.sum(jnp.where(sel, x2, 0.0), axis=1, keepdims=True)
        bcl = jnp.sum(jnp.where(sel, cls, 0), axis=1, keepdims=True)
        barea = (by2 - by1) * (bx2 - bx1)
        iy1 = jnp.maximum(by1, y1)
        ix1 = jnp.maximum(bx1, x1)
        iy2 = jnp.minimum(by2, y2)
        ix2 = jnp.minimum(bx2, x2)
        inter = jnp.maximum(iy2 - iy1, 0.0) * jnp.maximum(ix2 - ix1, 0.0)
        iou = inter / (barea + area - inter + 1e-8)
        live = jnp.where(iou > IOU_T, NEG, live)
        live = jnp.where(sel, NEG, live)
        slot = slot_lane == t
        osc = jnp.where(slot & valid, m, osc)
        oy1 = jnp.where(slot & valid, by1, oy1)
        ox1 = jnp.where(slot & valid, bx1, ox1)
        oy2 = jnp.where(slot & valid, by2, oy2)
        ox2 = jnp.where(slot & valid, bx2, ox2)
        ocl = jnp.where(slot & valid, bcl, ocl)
        nv = nv + valid.astype(jnp.int32)
        return live, osc, oy1, ox1, oy2, ox2, ocl, nv

    carry = (live0, zf, zf, zf, zf, zf, zi, nv0)
    _, osc, oy1, ox1, oy2, ox2, ocl, nv = lax.fori_loop(0, MAXDET, body, carry)
    osc_ref[...] = osc
    oy1_ref[...] = oy1
    ox1_ref[...] = ox1
    oy2_ref[...] = oy2
    ox2_ref[...] = ox2
    ocl_ref[...] = ocl
    ovd_ref[...] = jnp.broadcast_to(nv, (B, 128))


def kernel(boxes, scores):
    smax, cls = pl.pallas_call(
        _class_reduce_body,
        grid=(B, NCH),
        in_specs=[pl.BlockSpec((1, CH, C), lambda b, n: (b, n, 0))],
        out_specs=[
            pl.BlockSpec((1, CH, 1), lambda b, n: (b, n, 0)),
            pl.BlockSpec((1, CH, 1), lambda b, n: (b, n, 0)),
        ],
        out_shape=[
            jax.ShapeDtypeStruct((B, N, 1), jnp.float32),
            jax.ShapeDtypeStruct((B, N, 1), jnp.int32),
        ],
    )(scores)
    smax = smax[..., 0]
    cls = cls[..., 0]

    pad = NP - N
    smax_p = jnp.pad(smax, ((0, 0), (0, pad)), constant_values=NEG)
    cls_p = jnp.pad(cls, ((0, 0), (0, pad)))
    planes = jnp.moveaxis(boxes, -1, 0)
    planes = jnp.pad(planes, ((0, 0), (0, 0), (0, pad)))
    y1p, x1p, y2p, x2p = planes[0], planes[1], planes[2], planes[3]

    full = pl.BlockSpec((B, NP), lambda: (0, 0))
    outs = pl.pallas_call(
        _nms_body,
        grid=(),
        in_specs=[full] * 6,
        out_specs=[pl.BlockSpec((B, 128), lambda: (0, 0))] * 7,
        out_shape=[jax.ShapeDtypeStruct((B, 128), jnp.float32)] * 5
        + [jax.ShapeDtypeStruct((B, 128), jnp.int32)] * 2,
    )(smax_p, cls_p, y1p, x1p, y2p, x2p)
    osc, oy1, ox1, oy2, ox2, ocl, ovd = outs

    nmsed_scores = osc[:, :MAXDET]
    nmsed_boxes = jnp.stack(
        [oy1[:, :MAXDET], ox1[:, :MAXDET], oy2[:, :MAXDET], ox2[:, :MAXDET]],
        axis=-1,
    )
    nmsed_classes = ocl[:, :MAXDET]
    valid = ovd[:, 0]
    return nmsed_scores, nmsed_boxes, nmsed_classes, valid
